# baseline (speedup 1.0000x reference)
import jax
import jax.numpy as jnp
from jax import lax
from jax.experimental import pallas as pl
from jax.experimental.pallas import tpu as pltpu
from jax.experimental.pallas import tpu_sc as plsc

_NW = 32
_NWIN = 4
_BT = 512
_CP = 28
_NK = 7


def _sc_gather(tab_flat, idx4, e):
    n = idx4.shape[1]
    per_w = n // _NW
    win = per_w // _NWIN
    mesh = plsc.VectorSubcoreMesh(core_axis_name="core", subcore_axis_name="subcore")

    @pl.kernel(
        out_type=jax.ShapeDtypeStruct((n, 128), jnp.float32),
        mesh=mesh,
        scratch_types=[
            pltpu.VMEM((4, win), jnp.int32),
            pltpu.VMEM((4, win, e), jnp.float32),
            pltpu.SemaphoreType.DMA,
        ],
        compiler_params=pltpu.CompilerParams(use_tc_tiling_on_sc=False),
    )
    def k(tab_hbm, ci_hbm, o_hbm, idx_v, rows_v, sem):
        wid = lax.axis_index("subcore") * 2 + lax.axis_index("core")
        base = wid * per_w

        @pl.loop(0, _NWIN)
        def _(w):
            off = base + w * win
            pltpu.sync_copy(ci_hbm.at[:, pl.ds(off, win)], idx_v)
            copies = [
                pltpu.async_copy(tab_hbm.at[idx_v.at[q]], rows_v.at[q], sem)
                for q in range(4)
            ]
            for c in copies:
                c.wait()
            for q in range(4):
                pltpu.sync_copy(rows_v.at[q],
                                o_hbm.at[pl.ds(off, win), pl.ds(q * e, e)])

    return k(tab_flat, idx4)


def _mlp_body(num_ref, gath_ref, wW1n_ref, wW1e_ref, wb1_ref, wW2_ref,
              dW1n_ref, dW1e_ref, db1_ref, dW2_ref, db2_ref, dW3_ref,
              db3_ref, dW4_ref, cbias_ref, out_ref):
    fn = num_ref[...]
    fe = [
        gath_ref[pl.ds(k * _BT, _BT), :].astype(jnp.bfloat16)
        for k in range(_NK)
    ]
    h = jnp.dot(fn, wW1n_ref[...], preferred_element_type=jnp.float32)
    d = jnp.dot(fn, dW1n_ref[...], preferred_element_type=jnp.float32)
    for k in range(_NK):
        wk = wW1e_ref[pl.ds(k * 128, 128), :]
        dk = dW1e_ref[pl.ds(k * 128, 128), :]
        h = h + jnp.dot(fe[k], wk, preferred_element_type=jnp.float32)
        d = d + jnp.dot(fe[k], dk, preferred_element_type=jnp.float32)
    h = jnp.maximum(h + wb1_ref[...], 0.0)
    wide = jnp.dot(h.astype(jnp.bfloat16), wW2_ref[...],
                   preferred_element_type=jnp.float32)
    d = jnp.maximum(d + db1_ref[...], 0.0)
    d = jnp.maximum(
        jnp.dot(d, dW2_ref[...], preferred_element_type=jnp.float32) + db2_ref[...], 0.0)
    d = jnp.maximum(
        jnp.dot(d, dW3_ref[...], preferred_element_type=jnp.float32) + db3_ref[...], 0.0)
    deep = jnp.dot(d, dW4_ref[...], preferred_element_type=jnp.float32)
    out_ref[...] = wide + deep + cbias_ref[...]


def _mlp(num, gath, wW1n, wW1e, wb1, wW2, dW1n, dW1e, db1, dW2, db2, dW3,
         db3, dW4, cbias):
    b = num.shape[0]
    grid = (b // _BT,)
    gtile = gath.shape[0] // (b // _BT)
    full = lambda shape: pl.BlockSpec(shape, lambda i: (0, 0))
    return pl.pallas_call(
        _mlp_body,
        grid=grid,
        in_specs=[
            pl.BlockSpec((_BT, num.shape[1]), lambda i: (i, 0)),
            pl.BlockSpec((gtile, 128), lambda i: (i, 0)),
            full(wW1n.shape), full(wW1e.shape), full(wb1.shape),
            full(wW2.shape), full(dW1n.shape), full(dW1e.shape),
            full(db1.shape), full(dW2.shape), full(db2.shape),
            full(dW3.shape), full(db3.shape), full(dW4.shape),
            full(cbias.shape),
        ],
        out_specs=pl.BlockSpec((_BT, 1), lambda i: (i, 0)),
        out_shape=jax.ShapeDtypeStruct((b, 1), jnp.float32),
    )(num, gath, wW1n, wW1e, wb1, wW2, dW1n, dW1e, db1, dW2, db2, dW3,
      db3, dW4, cbias)


def _pad_rows(w, rows):
    return jnp.concatenate(
        [w, jnp.zeros((rows - w.shape[0],) + w.shape[1:], w.dtype)], axis=0)


def kernel(numerical_fields, categorical_fields, tables,
           wide_W1, wide_b1, wide_W2, wide_b2,
           deep_W1, deep_b1, deep_W2, deep_b2,
           deep_W3, deep_b3, deep_W4, deep_b4, bias):
    b, num = numerical_fields.shape
    cat, v, e = tables.shape
    nt = b // _BT
    offs = jnp.arange(cat, dtype=jnp.int32) * v
    idx28 = jnp.concatenate(
        [categorical_fields + offs[None, :],
         jnp.zeros((b, _CP - cat), jnp.int32)], axis=1)
    idx4 = (idx28.reshape(nt, _BT, _NK, 4)
            .transpose(3, 0, 2, 1).reshape(4, b * _NK))
    gath = _sc_gather(tables.reshape(cat * v, e), idx4, e)

    epad = _NK * 128
    cbias = (wide_b2 + deep_b4 + bias).reshape(1, 1)
    out = _mlp(
        numerical_fields, gath,
        wide_W1[:num], _pad_rows(wide_W1[num:], epad).astype(jnp.bfloat16),
        wide_b1.reshape(1, -1), wide_W2.astype(jnp.bfloat16),
        deep_W1[:num], _pad_rows(deep_W1[num:], epad).astype(jnp.bfloat16),
        deep_b1.reshape(1, -1), deep_W2,
        deep_b2.reshape(1, -1), deep_W3, deep_b3.reshape(1, -1), deep_W4,
        cbias)
    return out


# speedup vs baseline: 1.2200x; 1.2200x over previous
import jax
import jax.numpy as jnp
from jax import lax
from jax.experimental import pallas as pl
from jax.experimental.pallas import tpu as pltpu
from jax.experimental.pallas import tpu_sc as plsc

_NW = 32
_NWIN = 4
_BT = 512


def _sc_gather(tab_flat, idx_flat, e):
    n = idx_flat.shape[0]
    per_w = n // _NW
    win = per_w // _NWIN
    mesh = plsc.VectorSubcoreMesh(core_axis_name="core", subcore_axis_name="subcore")

    @pl.kernel(
        out_type=jax.ShapeDtypeStruct((n, e), jnp.float32),
        mesh=mesh,
        scratch_types=[
            pltpu.VMEM((win,), jnp.int32),
            pltpu.VMEM((win, e), jnp.float32),
            pltpu.SemaphoreType.DMA,
        ],
        compiler_params=pltpu.CompilerParams(use_tc_tiling_on_sc=False),
    )
    def k(tab_hbm, ci_hbm, o_hbm, idx_v, rows_v, sem):
        wid = lax.axis_index("subcore") * 2 + lax.axis_index("core")
        base = wid * per_w

        @pl.loop(0, _NWIN)
        def _(w):
            off = base + w * win
            pltpu.sync_copy(ci_hbm.at[pl.ds(off, win)], idx_v)
            pltpu.async_copy(tab_hbm.at[idx_v], rows_v, sem).wait()
            pltpu.sync_copy(rows_v, o_hbm.at[pl.ds(off, win)])

    return k(tab_flat, idx_flat)


def _mlp_body(num_ref, gath_ref, wW1n_ref, wW1e_ref, wb1_ref, wW2_ref,
              dW1n_ref, dW1e_ref, db1_ref, dW2_ref, db2_ref, dW3_ref,
              db3_ref, dW4_ref, cbias_ref, out_ref):
    fn = num_ref[...]
    fe = gath_ref[...].astype(jnp.bfloat16)
    h = jnp.dot(fn, wW1n_ref[...], preferred_element_type=jnp.float32)
    h = h + jnp.dot(fe, wW1e_ref[...], preferred_element_type=jnp.float32)
    h = jnp.maximum(h + wb1_ref[...], 0.0)
    wide = jnp.dot(h.astype(jnp.bfloat16), wW2_ref[...],
                   preferred_element_type=jnp.float32)
    d = jnp.dot(fn, dW1n_ref[...], preferred_element_type=jnp.float32)
    d = d + jnp.dot(fe, dW1e_ref[...], preferred_element_type=jnp.float32)
    d = jnp.maximum(d + db1_ref[...], 0.0)
    d = jnp.maximum(
        jnp.dot(d, dW2_ref[...], preferred_element_type=jnp.float32) + db2_ref[...], 0.0)
    d = jnp.maximum(
        jnp.dot(d, dW3_ref[...], preferred_element_type=jnp.float32) + db3_ref[...], 0.0)
    deep = jnp.dot(d, dW4_ref[...], preferred_element_type=jnp.float32)
    out_ref[...] = wide + deep + cbias_ref[...]


def _mlp(num, gath, wW1n, wW1e, wb1, wW2, dW1n, dW1e, db1, dW2, db2, dW3,
         db3, dW4, cbias):
    b = num.shape[0]
    grid = (b // _BT,)
    full = lambda shape: pl.BlockSpec(shape, lambda i: (0, 0))
    return pl.pallas_call(
        _mlp_body,
        grid=grid,
        in_specs=[
            pl.BlockSpec((_BT, num.shape[1]), lambda i: (i, 0)),
            pl.BlockSpec((_BT, gath.shape[1]), lambda i: (i, 0)),
            full(wW1n.shape), full(wW1e.shape), full(wb1.shape),
            full(wW2.shape), full(dW1n.shape), full(dW1e.shape),
            full(db1.shape), full(dW2.shape), full(db2.shape),
            full(dW3.shape), full(db3.shape), full(dW4.shape),
            full(cbias.shape),
        ],
        out_specs=pl.BlockSpec((_BT, 1), lambda i: (i, 0)),
        out_shape=jax.ShapeDtypeStruct((b, 1), jnp.float32),
    )(num, gath, wW1n, wW1e, wb1, wW2, dW1n, dW1e, db1, dW2, db2, dW3,
      db3, dW4, cbias)


def kernel(numerical_fields, categorical_fields, tables,
           wide_W1, wide_b1, wide_W2, wide_b2,
           deep_W1, deep_b1, deep_W2, deep_b2,
           deep_W3, deep_b3, deep_W4, deep_b4, bias):
    b, num = numerical_fields.shape
    cat, v, e = tables.shape
    offs = jnp.arange(cat, dtype=jnp.int32) * v
    idx_flat = (categorical_fields + offs[None, :]).reshape(b * cat)
    rows = _sc_gather(tables.reshape(cat * v, e), idx_flat, e)
    gath = rows.reshape(b, cat * e)

    cbias = (wide_b2 + deep_b4 + bias).reshape(1, 1)
    out = _mlp(
        numerical_fields, gath,
        wide_W1[:num], wide_W1[num:].astype(jnp.bfloat16),
        wide_b1.reshape(1, -1), wide_W2.astype(jnp.bfloat16),
        deep_W1[:num], deep_W1[num:].astype(jnp.bfloat16),
        deep_b1.reshape(1, -1), deep_W2,
        deep_b2.reshape(1, -1), deep_W3, deep_b3.reshape(1, -1), deep_W4,
        cbias)
    return out
